# baseline (speedup 1.0000x reference)
import jax
import jax.numpy as jnp
from jax import lax
from jax.experimental import pallas as pl
from jax.experimental.pallas import tpu as pltpu
from jax.experimental.pallas import tpu_sc as plsc

L = 16
D = 64
V = 1000000
S = 100000
B = 16384
NW = 32
BPW = B // NW
CHK = 128
NCK = BPW // CHK


def _stock_body(stk_hbm, ws2_hbm, sg_hbm, stk_idx, pair_s, rows_a, rows_b,
                s_rows, sem_a, sem_b):
    wid = lax.axis_index("s") * 2 + lax.axis_index("c")
    base = wid * BPW
    lane = lax.iota(jnp.int32, L)

    pltpu.sync_copy(stk_hbm.at[pl.ds(base, BPW)], stk_idx)
    for j in range(NCK):
        for v in range(CHK // L):
            pair_s[j, pl.ds(v * L, L)] = (
                stk_idx[pl.ds(j * CHK + v * L, L)] >> 1)

    handles = {}

    def issue(j):
        buf = rows_a if j % 2 == 0 else rows_b
        sem = sem_a if j % 2 == 0 else sem_b
        handles[j] = pltpu.async_copy(ws2_hbm.at[pair_s.at[j]], buf, sem)

    issue(0)
    issue(1)
    for j in range(NCK):
        buf = rows_a if j % 2 == 0 else rows_b
        handles[j].wait()
        def ext_d(d, carry):
            for v in range(CHK // L):
                ent = j * CHK + v * L
                cols = (stk_idx[pl.ds(ent, L)] & 1) * D + d
                vals = plsc.load_gather(buf, [v * L + lane, cols])
                s_rows[d, pl.ds(ent, L)] = vals
            return carry
        lax.fori_loop(0, D, ext_d, 0)
        if j + 2 < NCK:
            issue(j + 2)

    pltpu.sync_copy(s_rows, sg_hbm.at[wid])


def _inv_body(inv_hbm, w2_hbm, sg_hbm, out_hbm, inv_idx, pair_i,
              rows_a, rows_b, s_rows, out_v, sem_a, sem_b):
    wid = lax.axis_index("s") * 2 + lax.axis_index("c")
    base = wid * BPW
    lane = lax.iota(jnp.int32, L)

    pltpu.sync_copy(inv_hbm.at[pl.ds(base, BPW)], inv_idx)
    pltpu.sync_copy(sg_hbm.at[wid], s_rows)
    for j in range(NCK):
        for v in range(CHK // L):
            pair_i[j, pl.ds(v * L, L)] = (
                inv_idx[pl.ds(j * CHK + v * L, L)] >> 1)

    handles = {}

    def issue(j):
        buf = rows_a if j % 2 == 0 else rows_b
        sem = sem_a if j % 2 == 0 else sem_b
        handles[j] = pltpu.async_copy(w2_hbm.at[pair_i.at[j]], buf, sem)

    issue(0)
    issue(1)
    for j in range(NCK):
        buf = rows_a if j % 2 == 0 else rows_b
        handles[j].wait()
        for v in range(CHK // L):
            ent = j * CHK + v * L
            rows = v * L + lane
            hi = (inv_idx[pl.ds(ent, L)] & 1) * D
            acc0 = jnp.zeros((L,), jnp.float32)
            acc1 = jnp.zeros((L,), jnp.float32)
            for d in range(0, D, 2):
                vi0 = plsc.load_gather(buf, [rows, hi + d])
                s0 = s_rows[d, pl.ds(ent, L)]
                vi1 = plsc.load_gather(buf, [rows, hi + (d + 1)])
                s1 = s_rows[d + 1, pl.ds(ent, L)]
                acc0 = acc0 + vi0 * s0
                acc1 = acc1 + vi1 * s1
            out_v[pl.ds(ent, L)] = acc0 + acc1
        if j + 2 < NCK:
            issue(j + 2)

    pltpu.sync_copy(out_v, out_hbm.at[pl.ds(base, BPW)])


def kernel(investor, stock_positive, investor_train, W_investor, W_stock):
    del investor_train
    mesh = plsc.VectorSubcoreMesh(core_axis_name="c", subcore_axis_name="s")
    params = pltpu.CompilerParams(
        needs_layout_passes=False, use_tc_tiling_on_sc=True)

    k_stock = pl.kernel(
        _stock_body,
        out_type=jax.ShapeDtypeStruct((NW, D, BPW), jnp.float32),
        mesh=mesh,
        compiler_params=params,
        scratch_types=[
            pltpu.VMEM((BPW,), jnp.int32),
            pltpu.VMEM((NCK, CHK), jnp.int32),
            pltpu.VMEM((CHK, 2 * D), jnp.float32),
            pltpu.VMEM((CHK, 2 * D), jnp.float32),
            pltpu.VMEM((D, BPW), jnp.float32),
            pltpu.SemaphoreType.DMA,
            pltpu.SemaphoreType.DMA,
        ],
    )

    k_inv = pl.kernel(
        _inv_body,
        out_type=jax.ShapeDtypeStruct((B,), jnp.float32),
        mesh=mesh,
        compiler_params=params,
        scratch_types=[
            pltpu.VMEM((BPW,), jnp.int32),
            pltpu.VMEM((NCK, CHK), jnp.int32),
            pltpu.VMEM((CHK, 2 * D), jnp.float32),
            pltpu.VMEM((CHK, 2 * D), jnp.float32),
            pltpu.VMEM((D, BPW), jnp.float32),
            pltpu.VMEM((BPW,), jnp.float32),
            pltpu.SemaphoreType.DMA,
            pltpu.SemaphoreType.DMA,
        ],
    )

    z = (investor[0] & 0).astype(jnp.float32)
    w2 = (W_investor + z).reshape(V // 2, 2 * D)
    ws2 = (W_stock + z).reshape(S // 2, 2 * D)
    s_g = k_stock(stock_positive, ws2)
    return k_inv(investor, w2, s_g)


# speedup vs baseline: 1.0761x; 1.0761x over previous
import functools

import jax
import jax.numpy as jnp
from jax import lax
from jax.experimental import pallas as pl
from jax.experimental.pallas import tpu as pltpu
from jax.experimental.pallas import tpu_sc as plsc

L = 16
D = 64
V = 1000000
S = 100000
B = 16384
NW = 32
CAP = 768
GRPW = 384
NG = 82
VPW = V // NW
NCH = 16
CHW = B // NCH
TAILBASE = (V // 128) * 128
CLAMP = TAILBASE - GRPW
TAILN = V - TAILBASE


def _k2_body(inv_hbm, stk_hbm, wt_hbm, ws2_hbm, wtail_hbm, bout_hbm,
             rout_hbm, inv_buf, stk_buf, inv_list, stk_list, b_stage,
             pair_buf, stk_chunk, wtail_v, s_matT, slabs, res_v, cnt_s,
             sem, sem2):
    cid = lax.axis_index("c")
    sid = lax.axis_index("s")
    wid = sid * 2 + cid
    lo = wid * VPW
    lane = lax.iota(jnp.int32, L)
    zero16 = jnp.zeros((L,), jnp.int32)

    for k in range(CAP // L + 1):
        stk_list[pl.ds(k * L, L)] = zero16
        b_stage[pl.ds(k * L, L)] = zero16 - 1

    def scan_chunk(ch, cur):
        pltpu.sync_copy(inv_hbm.at[pl.ds(ch * CHW, CHW)], inv_buf)
        pltpu.sync_copy(stk_hbm.at[pl.ds(ch * CHW, CHW)], stk_buf)

        def scan_vreg(k, cur2):
            inv_v = inv_buf[pl.ds(k * L, L)]
            m = (inv_v >= lo) & (inv_v < lo + VPW)
            stk_v = stk_buf[pl.ds(k * L, L)]
            bvals = ch * CHW + k * L + lane
            plsc.store_compressed(inv_list.at[pl.ds(cur2, L)], inv_v, mask=m)
            plsc.store_compressed(stk_list.at[pl.ds(cur2, L)], stk_v, mask=m)
            plsc.store_compressed(b_stage.at[pl.ds(cur2, L)], bvals, mask=m)
            return cur2 + lax.reduce_sum_p.bind(m.astype(jnp.int32), axes=(0,))

        return lax.fori_loop(0, CHW // L, scan_vreg, cur)

    cnt = lax.fori_loop(0, NCH, scan_chunk, 0)
    cnt_s[0] = cnt
    cnt_s[1] = (cnt + L - 1) // L

    for j in range(CAP // 128):
        for v in range(128 // L):
            pair_buf[pl.ds(v * L, L)] = (
                stk_list[pl.ds(j * 128 + v * L, L)] >> 1)
        pltpu.async_copy(ws2_hbm.at[pair_buf], stk_chunk, sem2).wait()

        def ext_d(d, carry):
            for v in range(128 // L):
                ent = j * 128 + v * L
                halfsel = stk_list[pl.ds(ent, L)] & 1
                cols = halfsel * D + d
                vals = plsc.load_gather(stk_chunk,
                                        [v * L + lane, cols])
                s_matT[d, pl.ds(ent, L)] = vals
            return carry

        lax.fori_loop(0, D, ext_d, 0)

    c0 = lo // 128

    def src_for(g):
        base_raw = (c0 + 3 * g) * 128
        base = pl.multiple_of(jnp.minimum(base_raw, CLAMP), 128)
        return base_raw, base

    def issue(g, buf):
        _, base = src_for(g)
        return pltpu.async_copy(
            wt_hbm.at[pl.ds(0, D), pl.ds(base, GRPW)], slabs.at[buf], sem)

    issue(0, 1)
    pltpu.sync_copy(wtail_hbm, wtail_v)

    def group_fn(g, carry):
        buf = 1 - (g % 2)
        base_raw, base = src_for(g)
        pltpu.make_async_copy(
            wt_hbm.at[pl.ds(0, D), pl.ds(base, GRPW)],
            slabs.at[buf], sem).wait()

        @pl.when(g + 1 < NG)
        def _():
            issue(g + 1, 1 - ((g + 1) % 2))

        cnt_v = cnt_s[0]
        nv = cnt_s[1]

        def vreg_fn(k, carry2):
            e1 = inv_list[pl.ds(k * L, L)]
            pos = k * L + lane
            m = ((e1 >= base_raw) & (e1 < base_raw + GRPW)
                 & (e1 < TAILBASE) & (pos < cnt_v))
            anym = lax.reduce_max_p.bind(m.astype(jnp.int32), axes=(0,))

            @pl.when(anym > 0)
            def _():
                l_vec = e1 - base
                acc0 = jnp.zeros((L,), jnp.float32)
                acc1 = jnp.zeros((L,), jnp.float32)
                for d in range(0, D, 2):
                    bsel = jnp.full((L,), buf, jnp.int32)
                    v0 = plsc.load_gather(
                        slabs, [bsel, jnp.full((L,), d, jnp.int32), l_vec],
                        mask=m)
                    s0 = plsc.load_gather(
                        s_matT, [jnp.full((L,), d, jnp.int32), pos], mask=m)
                    v1 = plsc.load_gather(
                        slabs, [bsel, jnp.full((L,), d + 1, jnp.int32),
                                l_vec], mask=m)
                    s1 = plsc.load_gather(
                        s_matT, [jnp.full((L,), d + 1, jnp.int32), pos],
                        mask=m)
                    acc0 = acc0 + v0 * s0
                    acc1 = acc1 + v1 * s1
                plsc.store_scatter(res_v, [pos], acc0 + acc1, mask=m)

            return carry2

        lax.fori_loop(0, nv, vreg_fn, 0)
        return carry

    lax.fori_loop(0, NG, group_fn, 0)

    def tail_vreg(k, carry):
        e1 = inv_list[pl.ds(k * L, L)]
        pos = k * L + lane
        m = (e1 >= TAILBASE) & (pos < cnt_s[0])
        anym = lax.reduce_max_p.bind(m.astype(jnp.int32), axes=(0,))

        @pl.when(anym > 0)
        def _():
            t = e1 - TAILBASE
            rowv = t >> 1
            half = t & 1
            acc = jnp.zeros((L,), jnp.float32)
            for d in range(D):
                v_d = plsc.load_gather(
                    wtail_v, [rowv, half * D + d], mask=m)
                s_d = plsc.load_gather(
                    s_matT, [jnp.full((L,), d, jnp.int32), pos], mask=m)
                acc = acc + v_d * s_d
            plsc.store_scatter(res_v, [pos], acc, mask=m)

        return carry

    lax.fori_loop(0, cnt_s[1], tail_vreg, 0)

    pltpu.sync_copy(b_stage.at[pl.ds(0, CAP)], bout_hbm.at[wid])
    pltpu.sync_copy(res_v, rout_hbm.at[wid])


def _k3_body(bout_hbm, rout_hbm, out_hbm, pairs_b, pairs_r, out_local, sem):
    wid = lax.axis_index("s") * 2 + lax.axis_index("c")
    pltpu.sync_copy(bout_hbm, pairs_b)
    pltpu.sync_copy(rout_hbm, pairs_r)

    def scatter_row(w, carry):
        def scatter_vreg(k, carry2):
            b_v = pairs_b[w, pl.ds(k * L, L)]
            r_v = pairs_r[w, pl.ds(k * L, L)]
            m = b_v >= 0
            plsc.store_scatter(out_local, [b_v], r_v, mask=m)
            return carry2
        return lax.fori_loop(0, CAP // L, scatter_vreg, carry)

    lax.fori_loop(0, NW, scatter_row, 0)
    sl = B // NW
    pltpu.sync_copy(out_local.at[pl.ds(wid * sl, sl)],
                    out_hbm.at[pl.ds(wid * sl, sl)])


def kernel(investor, stock_positive, investor_train, W_investor, W_stock):
    del investor_train
    mesh = plsc.VectorSubcoreMesh(core_axis_name="c", subcore_axis_name="s")

    k2 = pl.kernel(
        _k2_body,
        out_type=(jax.ShapeDtypeStruct((NW, CAP), jnp.int32),
                  jax.ShapeDtypeStruct((NW, CAP), jnp.float32)),
        mesh=mesh,
        compiler_params=pltpu.CompilerParams(
            needs_layout_passes=False, use_tc_tiling_on_sc=True),
        scratch_types=[
            pltpu.VMEM((CHW,), jnp.int32),
            pltpu.VMEM((CHW,), jnp.int32),
            pltpu.VMEM((CAP + L,), jnp.int32),
            pltpu.VMEM((CAP + L,), jnp.int32),
            pltpu.VMEM((CAP + L,), jnp.int32),
            pltpu.VMEM((128,), jnp.int32),
            pltpu.VMEM((128, 2 * D), jnp.float32),
            pltpu.VMEM((TAILN // 2, 2 * D), jnp.float32),
            pltpu.VMEM((D, CAP), jnp.float32),
            pltpu.VMEM((2, D, GRPW), jnp.float32),
            pltpu.VMEM((CAP,), jnp.float32),
            pltpu.SMEM((8,), jnp.int32),
            pltpu.SemaphoreType.DMA,
            pltpu.SemaphoreType.DMA,
        ],
    )

    k3 = pl.kernel(
        _k3_body,
        out_type=jax.ShapeDtypeStruct((B,), jnp.float32),
        mesh=mesh,
        compiler_params=pltpu.CompilerParams(needs_layout_passes=False),
        scratch_types=[
            pltpu.VMEM((NW, CAP), jnp.int32),
            pltpu.VMEM((NW, CAP), jnp.float32),
            pltpu.VMEM((B,), jnp.float32),
            pltpu.SemaphoreType.DMA,
        ],
    )

    wt = W_investor.T
    ws2 = W_stock.reshape(S // 2, 2 * D)
    wtail = W_investor[TAILBASE:].reshape(TAILN // 2, 2 * D)
    bout, rout = k2(investor, stock_positive, wt, ws2, wtail)
    return k3(bout, rout)
